# baseline (speedup 1.0000x reference)
import functools
import math

import jax
import jax.numpy as jnp
from jax import lax
from jax.experimental import pallas as pl
from jax.experimental.pallas import tpu as pltpu
from jax.experimental.pallas import tpu_sc as plsc

D_MODEL = 64
SCALE = math.sqrt(D_MODEL)
LANES = 128

_info = plsc.get_sparse_core_info()
_NC, _NS, _L = _info.num_cores, _info.num_subcores, _info.num_lanes
_NW = _NC * _NS


def _make_gather(V: int, J: int, I: int, D: int):
    assert I == _NW * LANES
    mesh = plsc.VectorSubcoreMesh(core_axis_name="c", subcore_axis_name="s")

    @functools.partial(
        pl.kernel,
        mesh=mesh,
        out_type=jax.ShapeDtypeStruct((J, D, I), jnp.float32),
        scratch_types=[
            pltpu.VMEM((J, LANES), jnp.int32),
            [pltpu.VMEM((LANES, LANES), jnp.float32) for _ in range(2)],
            [pltpu.VMEM((D, LANES), jnp.float32) for _ in range(2)],
            [pltpu.SemaphoreType.DMA for _ in range(2)],
            [pltpu.SemaphoreType.DMA for _ in range(2)],
        ],
        compiler_params=pltpu.CompilerParams(
            use_tc_tiling_on_sc=True, needs_layout_passes=False),
    )
    def gather_scale(table_hbm, xt_hbm, out_hbm, idx_v, gbufs, obufs, gsems, osems):
        wid = lax.axis_index("s") * _NC + lax.axis_index("c")
        i0 = wid * LANES
        pltpu.sync_copy(xt_hbm.at[:, pl.ds(i0, LANES)], idx_v)

        def issue_gather(j, b):
            pltpu.async_copy(table_hbm.at[idx_v.at[j]], gbufs[b], gsems[b])

        def wait_gather(b):
            pltpu.make_async_copy(
                table_hbm.at[pl.ds(0, LANES)], gbufs[b], gsems[b]).wait()

        def issue_owrite(j, b):
            pltpu.async_copy(
                obufs[b], out_hbm.at[j, :, pl.ds(i0, LANES)], osems[b])

        def wait_owrite(b):
            pltpu.make_async_copy(
                obufs[b], out_hbm.at[0, :, pl.ds(0, LANES)], osems[b]).wait()

        issue_gather(0, 0)
        lane = lax.iota(jnp.int32, _L)

        def j_body(jj, _):
            for b in range(2):
                j = 2 * jj + b
                wait_gather(b)

                @pl.when(j + 1 < J)
                def _():
                    issue_gather(j + 1, 1 - b)

                @pl.when(j >= 2)
                def _():
                    wait_owrite(b)

                def s_body(s, _):
                    s16 = jnp.full((_L,), s, jnp.int32)
                    for k in range(D // _L):
                        v = gbufs[b][s, pl.ds(k * _L, _L)] * SCALE
                        plsc.store_scatter(obufs[b], [lane + k * _L, s16], v)
                    return 0

                lax.fori_loop(0, LANES, s_body, 0)
                issue_owrite(j, b)

            return 0

        lax.fori_loop(0, J // 2, j_body, 0)
        wait_owrite(0)
        wait_owrite(1)

    return gather_scale


def kernel(x, emb_weight):
    S0, S1 = x.shape
    V, D = emb_weight.shape
    table_p = jnp.pad(emb_weight, ((0, 0), (0, LANES - D)))
    xt = jnp.transpose(x)
    out_t = _make_gather(V, S1, S0, D)(table_p, xt)
    return jnp.transpose(out_t, (2, 0, 1))


# speedup vs baseline: 1.3473x; 1.3473x over previous
import functools
import math

import jax
import jax.numpy as jnp
from jax import lax
from jax.experimental import pallas as pl
from jax.experimental.pallas import tpu as pltpu
from jax.experimental.pallas import tpu_sc as plsc

D_MODEL = 64
SCALE = math.sqrt(D_MODEL)
LANES = 128

_info = plsc.get_sparse_core_info()
_NC, _NS, _L = _info.num_cores, _info.num_subcores, _info.num_lanes
_NW = _NC * _NS


def _make_gather(V: int, J: int, I: int, D: int):
    assert I == _NW * LANES
    mesh = plsc.VectorSubcoreMesh(core_axis_name="c", subcore_axis_name="s")

    @functools.partial(
        pl.kernel,
        mesh=mesh,
        out_type=jax.ShapeDtypeStruct((J, D, I), jnp.float32),
        scratch_types=[
            pltpu.VMEM((J, LANES), jnp.int32),
            [pltpu.VMEM((LANES, LANES), jnp.float32) for _ in range(2)],
            [pltpu.VMEM((D, LANES), jnp.float32) for _ in range(2)],
            [pltpu.SemaphoreType.DMA for _ in range(2)],
            [pltpu.SemaphoreType.DMA for _ in range(2)],
        ],
        compiler_params=pltpu.CompilerParams(
            use_tc_tiling_on_sc=True, needs_layout_passes=False),
    )
    def gather_scale(table_hbm, xt_hbm, out_hbm, idx_v, gbufs, obufs, gsems, osems):
        wid = lax.axis_index("s") * _NC + lax.axis_index("c")
        i0 = wid * LANES
        pltpu.sync_copy(xt_hbm.at[:, pl.ds(i0, LANES)], idx_v)

        def issue_gather(j, b):
            pltpu.async_copy(table_hbm.at[idx_v.at[j]], gbufs[b], gsems[b])

        def wait_gather(b):
            pltpu.make_async_copy(
                table_hbm.at[pl.ds(0, LANES)], gbufs[b], gsems[b]).wait()

        def issue_owrite(j, b):
            pltpu.async_copy(
                obufs[b], out_hbm.at[j, :, pl.ds(i0, LANES)], osems[b])

        def wait_owrite(b):
            pltpu.make_async_copy(
                obufs[b], out_hbm.at[0, :, pl.ds(0, LANES)], osems[b]).wait()

        issue_gather(0, 0)
        lane = lax.iota(jnp.int32, _L)
        rows = [lane + k * _L for k in range(D // _L)]

        def j_body(jj, _):
            for b in range(2):
                j = 2 * jj + b
                wait_gather(b)

                @pl.when(j + 1 < J)
                def _():
                    issue_gather(j + 1, 1 - b)

                @pl.when(j >= 2)
                def _():
                    wait_owrite(b)

                @plsc.parallel_loop(0, LANES, unroll=8)
                def s_body(s):
                    s16 = jnp.full((_L,), s, jnp.int32)
                    for k in range(D // _L):
                        v = gbufs[b][s, pl.ds(k * _L, _L)] * SCALE
                        plsc.store_scatter(obufs[b], [rows[k], s16], v)

                issue_owrite(j, b)

            return 0

        lax.fori_loop(0, J // 2, j_body, 0)
        wait_owrite(0)
        wait_owrite(1)

    return gather_scale


def kernel(x, emb_weight):
    S0, S1 = x.shape
    V, D = emb_weight.shape
    table_p = jnp.pad(emb_weight, ((0, 0), (0, LANES - D)))
    xt = jnp.transpose(x)
    out_t = _make_gather(V, S1, S0, D)(table_p, xt)
    return jnp.transpose(out_t, (2, 0, 1))


# speedup vs baseline: 1.4088x; 1.0456x over previous
import functools
import math

import jax
import jax.numpy as jnp
from jax import lax
from jax.experimental import pallas as pl
from jax.experimental.pallas import tpu as pltpu
from jax.experimental.pallas import tpu_sc as plsc

D_MODEL = 64
SCALE = math.sqrt(D_MODEL)
LANES = 128

_info = plsc.get_sparse_core_info()
_NC, _NS, _L = _info.num_cores, _info.num_subcores, _info.num_lanes
_NW = _NC * _NS


def _make_gather(V: int, J: int, I: int, D: int):
    assert I == _NW * LANES
    mesh = plsc.VectorSubcoreMesh(core_axis_name="c", subcore_axis_name="s")

    @functools.partial(
        pl.kernel,
        mesh=mesh,
        out_type=jax.ShapeDtypeStruct((J, D, I), jnp.float32),
        scratch_types=[
            pltpu.VMEM((J, LANES), jnp.int32),
            [pltpu.VMEM((LANES, LANES), jnp.float32) for _ in range(2)],
            [pltpu.VMEM((D, LANES), jnp.float32) for _ in range(2)],
            [pltpu.SemaphoreType.DMA for _ in range(2)],
            [pltpu.SemaphoreType.DMA for _ in range(2)],
        ],
        compiler_params=pltpu.CompilerParams(
            use_tc_tiling_on_sc=True, needs_layout_passes=False),
    )
    def gather_scale(table_hbm, xt_hbm, out_hbm, idx_v, gbufs, obufs, gsems, osems):
        wid = lax.axis_index("s") * _NC + lax.axis_index("c")
        i0 = wid * LANES
        pltpu.sync_copy(xt_hbm.at[:, pl.ds(i0, LANES)], idx_v)

        def issue_gather(j, b):
            pltpu.async_copy(table_hbm.at[idx_v.at[j]], gbufs[b], gsems[b])

        def wait_gather(b):
            pltpu.make_async_copy(
                table_hbm.at[pl.ds(0, LANES)], gbufs[b], gsems[b]).wait()

        def issue_owrite(j, b):
            pltpu.async_copy(
                obufs[b], out_hbm.at[j, :, pl.ds(i0, LANES)], osems[b])

        def wait_owrite(b):
            pltpu.make_async_copy(
                obufs[b], out_hbm.at[0, :, pl.ds(0, LANES)], osems[b]).wait()

        issue_gather(0, 0)
        lane = lax.iota(jnp.int32, _L)
        rows_sb = [sb * _L + lane for sb in range(LANES // _L)]

        def j_body(jj, _):
            for b in range(2):
                j = 2 * jj + b
                wait_gather(b)

                @pl.when(j + 1 < J)
                def _():
                    issue_gather(j + 1, 1 - b)

                @pl.when(j >= 2)
                def _():
                    wait_owrite(b)

                @plsc.parallel_loop(0, D, unroll=8)
                def d_body(d):
                    dcol = jnp.full((_L,), d, jnp.int32)
                    for sb in range(LANES // _L):
                        v = plsc.load_gather(gbufs[b], [rows_sb[sb], dcol])
                        obufs[b][d, pl.ds(sb * _L, _L)] = v * SCALE

                issue_owrite(j, b)

            return 0

        lax.fori_loop(0, J // 2, j_body, 0)
        wait_owrite(0)
        wait_owrite(1)

    return gather_scale


def kernel(x, emb_weight):
    S0, S1 = x.shape
    V, D = emb_weight.shape
    table_p = jnp.pad(emb_weight, ((0, 0), (0, LANES - D)))
    xt = jnp.transpose(x)
    out_t = _make_gather(V, S1, S0, D)(table_p, xt)
    return jnp.transpose(out_t, (2, 0, 1))


# speedup vs baseline: 1.4098x; 1.0007x over previous
import functools
import math

import jax
import jax.numpy as jnp
from jax import lax
from jax.experimental import pallas as pl
from jax.experimental.pallas import tpu as pltpu
from jax.experimental.pallas import tpu_sc as plsc

D_MODEL = 64
SCALE = math.sqrt(D_MODEL)
LANES = 128

_info = plsc.get_sparse_core_info()
_NC, _NS, _L = _info.num_cores, _info.num_subcores, _info.num_lanes
_NW = _NC * _NS


def _make_gather(V: int, J: int, I: int, D: int):
    assert I == _NW * LANES
    mesh = plsc.VectorSubcoreMesh(core_axis_name="c", subcore_axis_name="s")

    @functools.partial(
        pl.kernel,
        mesh=mesh,
        out_type=jax.ShapeDtypeStruct((J, D, I), jnp.float32),
        scratch_types=[
            pltpu.VMEM((J, LANES), jnp.int32),
            [pltpu.VMEM((LANES, LANES), jnp.float32) for _ in range(4)],
            [pltpu.VMEM((D, LANES), jnp.float32) for _ in range(2)],
            [pltpu.SemaphoreType.DMA for _ in range(4)],
            [pltpu.SemaphoreType.DMA for _ in range(2)],
        ],
        compiler_params=pltpu.CompilerParams(
            use_tc_tiling_on_sc=True, needs_layout_passes=False),
    )
    def gather_scale(table_hbm, xt_hbm, out_hbm, idx_v, gbufs, obufs, gsems, osems):
        wid = lax.axis_index("s") * _NC + lax.axis_index("c")
        i0 = wid * LANES
        pltpu.sync_copy(xt_hbm.at[:, pl.ds(i0, LANES)], idx_v)

        def issue_gather(j, b):
            pltpu.async_copy(table_hbm.at[idx_v.at[j]], gbufs[b], gsems[b])

        def wait_gather(b):
            pltpu.make_async_copy(
                table_hbm.at[pl.ds(0, LANES)], gbufs[b], gsems[b]).wait()

        def issue_owrite(j, b):
            pltpu.async_copy(
                obufs[b], out_hbm.at[j, :, pl.ds(i0, LANES)], osems[b])

        def wait_owrite(b):
            pltpu.make_async_copy(
                obufs[b], out_hbm.at[0, :, pl.ds(0, LANES)], osems[b]).wait()

        issue_gather(0, 0)
        issue_gather(1, 1)
        lane = lax.iota(jnp.int32, _L)
        rows_sb = [sb * _L + lane for sb in range(LANES // _L)]

        def j_body(jj, _):
            for b in range(4):
                j = 4 * jj + b
                ob = b % 2
                wait_gather(b)

                @pl.when(j + 2 < J)
                def _():
                    issue_gather(j + 2, (b + 2) % 4)

                @pl.when(j >= 2)
                def _():
                    wait_owrite(ob)

                @plsc.parallel_loop(0, D, unroll=8)
                def d_body(d):
                    dcol = jnp.full((_L,), d, jnp.int32)
                    for sb in range(LANES // _L):
                        v = plsc.load_gather(gbufs[b], [rows_sb[sb], dcol])
                        obufs[ob][d, pl.ds(sb * _L, _L)] = v * SCALE

                issue_owrite(j, ob)

            return 0

        lax.fori_loop(0, J // 4, j_body, 0)
        wait_owrite(0)
        wait_owrite(1)

    return gather_scale


def kernel(x, emb_weight):
    S0, S1 = x.shape
    V, D = emb_weight.shape
    table_p = jnp.pad(emb_weight, ((0, 0), (0, LANES - D)))
    xt = jnp.transpose(x)
    out_t = _make_gather(V, S1, S0, D)(table_p, xt)
    return jnp.transpose(out_t, (2, 0, 1))
